# baseline (speedup 1.0000x reference)
import functools

import jax
import jax.numpy as jnp
from jax import lax
from jax.experimental import pallas as pl
from jax.experimental.pallas import tpu as pltpu
from jax.experimental.pallas import tpu_sc as plsc

_NUM_CORES = 2
_NUM_SUBCORES = 16
_NW = _NUM_CORES * _NUM_SUBCORES
_CHUNK = 128


def _argmin_body(x_ref, w_ref, idx_ref):
    xb = x_ref[0]
    w = w_ref[...]
    s = lax.dot_general(w, xb, (((1,), (0,)), ((), ())),
                        preferred_element_type=jnp.float32)
    b2 = jnp.sum(w * w, axis=1, keepdims=True)
    a2 = jnp.sum(xb * xb, axis=0, keepdims=True)
    d2 = jnp.maximum(a2 + b2 - 2.0 * s, 0.0)
    dist = jnp.sqrt(d2)
    n = xb.shape[1]
    idx = jnp.argmin(dist, axis=0).astype(jnp.int32)
    idx_ref[...] = idx.reshape(n // 128, 128)


def _nearest_indices(x_r, w):
    b, c, n = x_r.shape
    k = w.shape[0]
    rows_per_b = n // 128
    return pl.pallas_call(
        _argmin_body,
        grid=(b,),
        in_specs=[
            pl.BlockSpec((1, c, n), lambda i: (i, 0, 0)),
            pl.BlockSpec((k, c), lambda i: (0, 0)),
        ],
        out_specs=pl.BlockSpec((rows_per_b, 128), lambda i: (i, 0)),
        out_shape=jax.ShapeDtypeStruct((b * rows_per_b, 128), jnp.int32),
    )(x_r, w)


def _make_sc_gather(rows, d):
    per_w = rows // _NW
    n_ch = per_w // _CHUNK
    mesh = plsc.VectorSubcoreMesh(core_axis_name="c", subcore_axis_name="s")

    @functools.partial(
        pl.kernel,
        mesh=mesh,
        out_type=jax.ShapeDtypeStruct((rows, d), jnp.float32),
        compiler_params=pltpu.CompilerParams(use_tc_tiling_on_sc=False),
        scratch_types=[
            pltpu.VMEM((n_ch, _CHUNK), jnp.int32),
            pltpu.VMEM((per_w, d), jnp.float32),
            pltpu.SemaphoreType.DMA,
        ],
    )
    def gather_kernel(table_hbm, idx_hbm, out_hbm, idx_v, rows_v, sem):
        wid = lax.axis_index("s") * _NUM_CORES + lax.axis_index("c")
        base = wid * per_w
        pltpu.sync_copy(idx_hbm.at[pl.ds(wid * n_ch, n_ch)], idx_v)
        copies = []
        for j in range(n_ch):
            copies.append(pltpu.async_copy(
                table_hbm.at[idx_v.at[j]],
                rows_v.at[pl.ds(j * _CHUNK, _CHUNK)],
                sem,
            ))
        for cp in copies:
            cp.wait()
        pltpu.sync_copy(rows_v, out_hbm.at[pl.ds(base, per_w)])

    return gather_kernel


def kernel(x, W):
    b, c, h, w_sp = x.shape
    n = h * w_sp
    rows = b * n
    x_r = x.reshape(b, c, n)
    idx = _nearest_indices(x_r, W)
    quantized = _make_sc_gather(rows, c)(W, idx)
    return quantized.reshape(x.shape)


# speedup vs baseline: 1.0243x; 1.0243x over previous
import functools

import jax
import jax.numpy as jnp
from jax import lax
from jax.experimental import pallas as pl
from jax.experimental.pallas import tpu as pltpu
from jax.experimental.pallas import tpu_sc as plsc

_NUM_CORES = 2
_NUM_SUBCORES = 16
_NW = _NUM_CORES * _NUM_SUBCORES
_CHUNK = 128


def _argmin_body(x_ref, w_ref, idx_ref, w2_ref, b2_ref):
    k, n = w_ref.shape[0], x_ref.shape[2]
    @pl.when(pl.program_id(0) == 0)
    def _():
        w = w_ref[...]
        w2_ref[...] = w * -2.0
        b2_ref[...] = jnp.sum(w * w, axis=1, keepdims=True)

    xb = x_ref[0]
    s2 = lax.dot_general(w2_ref[...], xb, (((1,), (0,)), ((), ())),
                         preferred_element_type=jnp.float32)
    a2 = jnp.sum(xb * xb, axis=0, keepdims=True)
    d2 = jnp.maximum((a2 + b2_ref[...]) + s2, 0.0)
    dist = jnp.sqrt(d2)
    idx = jnp.argmin(dist, axis=0).astype(jnp.int32)
    idx_ref[...] = idx.reshape(n // 128, 128)


def _nearest_indices(x_r, w):
    b, c, n = x_r.shape
    k = w.shape[0]
    rows_per_b = n // 128
    return pl.pallas_call(
        _argmin_body,
        grid=(b,),
        in_specs=[
            pl.BlockSpec((1, c, n), lambda i: (i, 0, 0)),
            pl.BlockSpec((k, c), lambda i: (0, 0)),
        ],
        out_specs=pl.BlockSpec((rows_per_b, 128), lambda i: (i, 0)),
        out_shape=jax.ShapeDtypeStruct((b * rows_per_b, 128), jnp.int32),
        scratch_shapes=[
            pltpu.VMEM((k, c), jnp.float32),
            pltpu.VMEM((k, 1), jnp.float32),
        ],
    )(x_r, w)


def _make_sc_gather(rows, d):
    per_w = rows // _NW
    n_ch = per_w // _CHUNK
    mesh = plsc.VectorSubcoreMesh(core_axis_name="c", subcore_axis_name="s")

    @functools.partial(
        pl.kernel,
        mesh=mesh,
        out_type=jax.ShapeDtypeStruct((rows, d), jnp.float32),
        compiler_params=pltpu.CompilerParams(use_tc_tiling_on_sc=False),
        scratch_types=[
            pltpu.VMEM((n_ch, _CHUNK), jnp.int32),
            pltpu.VMEM((per_w, d), jnp.float32),
            pltpu.SemaphoreType.DMA,
        ],
    )
    def gather_kernel(table_hbm, idx_hbm, out_hbm, idx_v, rows_v, sem):
        wid = lax.axis_index("s") * _NUM_CORES + lax.axis_index("c")
        base = wid * per_w
        pltpu.sync_copy(idx_hbm.at[pl.ds(wid * n_ch, n_ch)], idx_v)
        copies = []
        for j in range(n_ch):
            copies.append(pltpu.async_copy(
                table_hbm.at[idx_v.at[j]],
                rows_v.at[pl.ds(j * _CHUNK, _CHUNK)],
                sem,
            ))
        for cp in copies:
            cp.wait()
        pltpu.sync_copy(rows_v, out_hbm.at[pl.ds(base, per_w)])

    return gather_kernel


def kernel(x, W):
    b, c, h, w_sp = x.shape
    n = h * w_sp
    rows = b * n
    x_r = x.reshape(b, c, n)
    idx = _nearest_indices(x_r, W)
    quantized = _make_sc_gather(rows, c)(W, idx)
    return quantized.reshape(x.shape)


# speedup vs baseline: 1.3049x; 1.2739x over previous
import functools

import jax
import jax.numpy as jnp
from jax import lax
from jax.experimental import pallas as pl
from jax.experimental.pallas import tpu as pltpu
from jax.experimental.pallas import tpu_sc as plsc

_NUM_CORES = 2
_NUM_SUBCORES = 16
_NW = _NUM_CORES * _NUM_SUBCORES
_CHUNK = 128


def _argmin_body(x_ref, wt_ref, idx_ref, w2t_ref, b2_ref):
    n = x_ref.shape[1]
    @pl.when(pl.program_id(0) == 0)
    def _():
        wt = wt_ref[...]
        w2t_ref[...] = wt * -2.0
        b2_ref[...] = jnp.sum(wt * wt, axis=0, keepdims=True)

    xn = x_ref[0]
    s2 = lax.dot_general(xn, w2t_ref[...], (((1,), (0,)), ((), ())),
                         preferred_element_type=jnp.float32)
    a2 = jnp.sum(xn * xn, axis=1, keepdims=True)
    d2 = jnp.maximum((a2 + b2_ref[...]) + s2, 0.0)
    dist = jnp.sqrt(d2)
    idx = jnp.argmin(dist, axis=1).astype(jnp.int32)
    idx_ref[...] = idx.reshape(n // 128, 128)


def _nearest_indices(x_rows, wt):
    b, n, c = x_rows.shape
    k = wt.shape[1]
    rows_per_b = n // 128
    return pl.pallas_call(
        _argmin_body,
        grid=(b,),
        in_specs=[
            pl.BlockSpec((1, n, c), lambda i: (i, 0, 0)),
            pl.BlockSpec((c, k), lambda i: (0, 0)),
        ],
        out_specs=pl.BlockSpec((rows_per_b, 128), lambda i: (i, 0)),
        out_shape=jax.ShapeDtypeStruct((b * rows_per_b, 128), jnp.int32),
        scratch_shapes=[
            pltpu.VMEM((c, k), jnp.float32),
            pltpu.VMEM((1, k), jnp.float32),
        ],
    )(x_rows, wt)


def _make_sc_gather(rows):
    per_w = rows // _NW
    n_ch = per_w // _CHUNK
    mesh = plsc.VectorSubcoreMesh(core_axis_name="c", subcore_axis_name="s")

    @functools.partial(
        pl.kernel,
        mesh=mesh,
        out_type=jax.ShapeDtypeStruct((rows, 64), jnp.float32),
        compiler_params=pltpu.CompilerParams(use_tc_tiling_on_sc=False),
        scratch_types=[
            pltpu.VMEM((n_ch, _CHUNK), jnp.int32),
            pltpu.VMEM((per_w, 64), jnp.float32),
            pltpu.SemaphoreType.DMA,
        ],
    )
    def gather_kernel(table_hbm, idx_hbm, out_hbm, idx_v, rows_v, sem):
        wid = lax.axis_index("s") * _NUM_CORES + lax.axis_index("c")
        base = wid * per_w
        pltpu.sync_copy(idx_hbm.at[pl.ds(wid * n_ch, n_ch)], idx_v)
        copies = []
        for j in range(n_ch):
            copies.append(pltpu.async_copy(
                table_hbm.at[idx_v.at[j]],
                rows_v.at[pl.ds(j * _CHUNK, _CHUNK)],
                sem,
            ))
        for cp in copies:
            cp.wait()
        pltpu.sync_copy(rows_v, out_hbm.at[pl.ds(base, per_w)])

    return gather_kernel


def kernel(x, W):
    b, c, h, w_sp = x.shape
    n = h * w_sp
    rows = b * n
    x_rows = jnp.transpose(x, (0, 2, 3, 1)).reshape(b, n, c)
    wt = W.T
    idx = _nearest_indices(x_rows, wt)
    quantized = _make_sc_gather(rows)(W, idx)
    return jnp.transpose(quantized.reshape(b, h, w_sp, c), (0, 3, 1, 2))
